# baseline (speedup 1.0000x reference)
import functools

import jax
import jax.numpy as jnp
from jax import lax
from jax.experimental import pallas as pl
from jax.experimental.pallas import tpu as pltpu, tpu_sc as plsc

N = 10000
E = 320000
D = 128

NC = 2
NS = 16
NW = NC * NS

CHUNK = 128
SEG = 40
K = 2 * SEG
K_DEG = 80
TOT_ROWS = NW * K
E_PAD = TOT_ROWS * CHUNK
N_PAD = 10240
Z = N_PAD // NS

def _sc_degree_body(dst_hbm, out0_hbm, out1_hbm, dst_v, ones_v, zbuf_v,
                    acc_sh, sem):
    c = lax.axis_index("c")
    s = lax.axis_index("s")
    w = c * NS + s

    @pl.loop(0, Z // 16)
    def _(i):
        zbuf_v[pl.ds(i * 16, 16)] = jnp.zeros((16,), jnp.float32)

    @pl.loop(0, CHUNK // 16)
    def _(i):
        ones_v[pl.ds(i * 16, 16)] = jnp.ones((16,), jnp.float32)

    z0 = pl.multiple_of(s * Z, 8)
    pltpu.sync_copy(zbuf_v, acc_sh.at[pl.ds(z0, Z)])
    plsc.subcore_barrier()

    pltpu.sync_copy(dst_hbm.at[pl.ds(w * (K_DEG * CHUNK), K_DEG * CHUNK)], dst_v)

    @pl.loop(0, K_DEG)
    def _(j):
        pltpu.sync_copy(ones_v, acc_sh.at[dst_v.at[pl.ds(j * CHUNK, CHUNK)]],
                        add=True)

    plsc.subcore_barrier()

    @pl.when(c == 0)
    def _():
        pltpu.sync_copy(acc_sh.at[pl.ds(z0, Z)], out0_hbm.at[pl.ds(z0, Z)])

    @pl.when(c == 1)
    def _():
        pltpu.sync_copy(acc_sh.at[pl.ds(z0, Z)], out1_hbm.at[pl.ds(z0, Z)])


def _sc_aggregate_body(g_hbm, src_hbm, dst_hbm, out0_hbm, out1_hbm,
                       src_v, dst_v, rows_a, rows_b, acc_sh,
                       sem_a, sem_b):
    c = lax.axis_index("c")
    s = lax.axis_index("s")
    w = c * NS + s

    @pl.loop(0, CHUNK)
    def _(i):
        for u in range(D // 16):
            rows_a[i, pl.ds(u * 16, 16)] = jnp.zeros((16,), jnp.float32)

    @pl.loop(0, Z // CHUNK)
    def _(i):
        pltpu.sync_copy(
            rows_a,
            acc_sh.at[pl.ds(pl.multiple_of(s * Z + i * CHUNK, 8), CHUNK)])

    plsc.subcore_barrier()

    def src_at(j):
        return src_v.at[pl.ds(j * CHUNK, CHUNK)]

    def dst_at(j):
        return dst_v.at[pl.ds(j * CHUNK, CHUNK)]

    for t in range(K // SEG):
        off = pl.multiple_of(w * K + t * SEG, 8)
        pltpu.sync_copy(src_hbm.at[pl.ds(off * CHUNK, SEG * CHUNK)], src_v)
        pltpu.sync_copy(dst_hbm.at[pl.ds(off * CHUNK, SEG * CHUNK)], dst_v)
        pltpu.async_copy(g_hbm.at[src_at(0)], rows_a, sem_a)

        @pl.loop(0, SEG, step=2)
        def _(j):
            pltpu.async_copy(g_hbm.at[src_at(j + 1)], rows_b, sem_b)
            pltpu.make_async_copy(g_hbm.at[src_at(j)], rows_a, sem_a).wait()
            pltpu.sync_copy(rows_a, acc_sh.at[dst_at(j)], add=True)

            @pl.when(j + 2 < SEG)
            def _():
                pltpu.async_copy(g_hbm.at[src_at(j + 2)], rows_a, sem_a)

            pltpu.make_async_copy(g_hbm.at[src_at(j + 1)], rows_b, sem_b).wait()
            pltpu.sync_copy(rows_b, acc_sh.at[dst_at(j + 1)], add=True)

    plsc.subcore_barrier()
    z0 = pl.multiple_of(s * Z, 8)

    @pl.when(c == 0)
    def _():
        pltpu.sync_copy(acc_sh.at[pl.ds(z0, Z)], out0_hbm.at[pl.ds(z0, Z)])

    @pl.when(c == 1)
    def _():
        pltpu.sync_copy(acc_sh.at[pl.ds(z0, Z)], out1_hbm.at[pl.ds(z0, Z)])


@functools.cache
def _build_sc_kernels():
    mesh = plsc.VectorSubcoreMesh(core_axis_name="c", subcore_axis_name="s",
                                  num_cores=NC, num_subcores=NS)
    deg = pl.kernel(
        _sc_degree_body,
        out_type=[jax.ShapeDtypeStruct((N_PAD,), jnp.float32),
                  jax.ShapeDtypeStruct((N_PAD,), jnp.float32)],
        mesh=mesh,
        scratch_types=[
            pltpu.VMEM((K_DEG * CHUNK,), jnp.int32),
            pltpu.VMEM((CHUNK,), jnp.float32),
            pltpu.VMEM((Z,), jnp.float32),
            pltpu.VMEM_SHARED((N_PAD,), jnp.float32),
            pltpu.SemaphoreType.DMA,
        ],
    )
    agg = pl.kernel(
        _sc_aggregate_body,
        out_type=[jax.ShapeDtypeStruct((N_PAD, D), jnp.float32),
                  jax.ShapeDtypeStruct((N_PAD, D), jnp.float32)],
        mesh=mesh,
        scratch_types=[
            pltpu.VMEM((SEG * CHUNK,), jnp.int32),
            pltpu.VMEM((SEG * CHUNK,), jnp.int32),
            pltpu.VMEM((CHUNK, D), jnp.float32),
            pltpu.VMEM((CHUNK, D), jnp.float32),
            pltpu.VMEM_SHARED((N_PAD, D), jnp.float32),
            pltpu.SemaphoreType.DMA,
            pltpu.SemaphoreType.DMA,
        ],
    )
    return deg, agg


def _sc_degree(dst_p):
    return _build_sc_kernels()[0](dst_p)


def _sc_aggregate(g, src_p, dst_p):
    return _build_sc_kernels()[1](g, src_p, dst_p)


BLK = 1000


def _tc1_body(x_ref, w_ref, p0_ref, p1_ref, g_ref, dis_ref):
    deg = p0_ref[...] + p1_ref[...] + 1.0
    dis = lax.rsqrt(deg)
    h = jnp.dot(x_ref[...], w_ref[...], preferred_element_type=jnp.float32)
    g_ref[...] = h * dis
    dis_ref[...] = dis


def _tc2_body(a0_ref, a1_ref, g_ref, dis_ref, b_ref, w_ref, g2_ref):
    dis = dis_ref[...]
    z = (a0_ref[...] + a1_ref[...] + g_ref[...]) * dis + b_ref[...]
    z = jnp.maximum(z, 0.0)
    h = jnp.dot(z, w_ref[...], preferred_element_type=jnp.float32)
    g2_ref[...] = h * dis


def _tc3_body(a0_ref, a1_ref, g_ref, dis_ref, b_ref, o_ref):
    t = (a0_ref[...] + a1_ref[...] + g_ref[...]) * dis_ref[...] + b_ref[...]
    m = jnp.max(t, axis=1, keepdims=True)
    lse = jnp.log(jnp.sum(jnp.exp(t - m), axis=1, keepdims=True))
    o_ref[...] = t - m - lse


def _row_spec(width):
    return pl.BlockSpec((BLK, width), lambda i: (i, 0))


def _full_spec(shape):
    return pl.BlockSpec(shape, lambda i: tuple(0 for _ in shape))


def _build_tc(interpret=False):
    tc1 = pl.pallas_call(
        _tc1_body,
        grid=(N // BLK,),
        in_specs=[_row_spec(D), _full_spec((D, D)), _row_spec(1), _row_spec(1)],
        out_specs=[_row_spec(D), _row_spec(1)],
        out_shape=[jax.ShapeDtypeStruct((N, D), jnp.float32),
                   jax.ShapeDtypeStruct((N, 1), jnp.float32)],
        interpret=interpret,
    )
    tc2 = pl.pallas_call(
        _tc2_body,
        grid=(N // BLK,),
        in_specs=[_row_spec(D), _row_spec(D), _row_spec(D), _row_spec(1),
                  _full_spec((1, D)), _full_spec((D, D))],
        out_specs=_row_spec(D),
        out_shape=jax.ShapeDtypeStruct((N, D), jnp.float32),
        interpret=interpret,
    )
    tc3 = pl.pallas_call(
        _tc3_body,
        grid=(N // BLK,),
        in_specs=[_row_spec(D), _row_spec(D), _row_spec(D), _row_spec(1),
                  _full_spec((1, D))],
        out_specs=_row_spec(D),
        out_shape=jax.ShapeDtypeStruct((N, D), jnp.float32),
        interpret=interpret,
    )
    return tc1, tc2, tc3


_tc1, _tc2, _tc3 = _build_tc()


def kernel(x, edge_index, W1, b1, W2, b2):
    src = edge_index[0].astype(jnp.int32)
    dst = edge_index[1].astype(jnp.int32)
    pad = E_PAD - E
    pad_idx = jnp.arange(pad, dtype=jnp.int32)
    src_pad = pad_idx % N
    dst_pad = N + pad_idx % (N_PAD - N)
    src_p = jnp.concatenate([src, src_pad])
    dst_p = jnp.concatenate([dst, dst_pad])

    p0, p1 = _sc_degree(dst_p)

    g1, dis = _tc1(x, W1, p0[:, None], p1[:, None])

    a0, a1 = _sc_aggregate(g1, src_p, dst_p)
    g2 = _tc2(a0, a1, g1, dis, b1[None, :], W2)

    a0, a1 = _sc_aggregate(g2, src_p, dst_p)
    return _tc3(a0, a1, g2, dis, b2[None, :])


# speedup vs baseline: 1.0022x; 1.0022x over previous
import functools

import jax
import jax.numpy as jnp
import numpy as np
from jax import lax
from jax.experimental import pallas as pl
from jax.experimental.pallas import tpu as pltpu, tpu_sc as plsc

N = 10000
E = 320000
D = 128

NC = 2
NS = 16
NW = NC * NS

CHUNK = 128
SEG = 40
K = 2 * SEG
K_DEG = 80
TOT_ROWS = NW * K
E_PAD = TOT_ROWS * CHUNK
N_PAD = 10240
Z = N_PAD // NS

def _sc_degree_body(dst_hbm, out0_hbm, out1_hbm, dst_v, ones_v, zbuf_v,
                    acc_sh, sem):
    c = lax.axis_index("c")
    s = lax.axis_index("s")
    w = c * NS + s

    @pl.loop(0, Z // 16)
    def _(i):
        zbuf_v[pl.ds(i * 16, 16)] = jnp.zeros((16,), jnp.float32)

    @pl.loop(0, CHUNK // 16)
    def _(i):
        ones_v[pl.ds(i * 16, 16)] = jnp.ones((16,), jnp.float32)

    z0 = pl.multiple_of(s * Z, 8)
    pltpu.sync_copy(zbuf_v, acc_sh.at[pl.ds(z0, Z)])
    plsc.subcore_barrier()

    pltpu.sync_copy(dst_hbm.at[pl.ds(w * (K_DEG * CHUNK), K_DEG * CHUNK)], dst_v)

    @pl.loop(0, K_DEG)
    def _(j):
        pltpu.sync_copy(ones_v, acc_sh.at[dst_v.at[pl.ds(j * CHUNK, CHUNK)]],
                        add=True)

    plsc.subcore_barrier()

    @pl.when(c == 0)
    def _():
        pltpu.sync_copy(acc_sh.at[pl.ds(z0, Z)], out0_hbm.at[pl.ds(z0, Z)])

    @pl.when(c == 1)
    def _():
        pltpu.sync_copy(acc_sh.at[pl.ds(z0, Z)], out1_hbm.at[pl.ds(z0, Z)])


def _sc_aggregate_body(g_hbm, src_hbm, dst_hbm, out0_hbm, out1_hbm,
                       src_v, dst_v, rows_a, rows_b, acc_sh,
                       sem_a, sem_b):
    c = lax.axis_index("c")
    s = lax.axis_index("s")
    w = c * NS + s

    @pl.loop(0, CHUNK)
    def _(i):
        for u in range(D // 16):
            rows_a[i, pl.ds(u * 16, 16)] = jnp.zeros((16,), jnp.float32)

    @pl.loop(0, Z // CHUNK)
    def _(i):
        pltpu.sync_copy(
            rows_a,
            acc_sh.at[pl.ds(pl.multiple_of(s * Z + i * CHUNK, 8), CHUNK)])

    plsc.subcore_barrier()

    def src_at(j):
        return src_v.at[pl.ds(j * CHUNK, CHUNK)]

    def dst_at(j):
        return dst_v.at[pl.ds(j * CHUNK, CHUNK)]

    for t in range(K // SEG):
        off = pl.multiple_of(w * K + t * SEG, 8)
        pltpu.sync_copy(src_hbm.at[pl.ds(off * CHUNK, SEG * CHUNK)], src_v)
        pltpu.sync_copy(dst_hbm.at[pl.ds(off * CHUNK, SEG * CHUNK)], dst_v)
        pltpu.async_copy(g_hbm.at[src_at(0)], rows_a, sem_a)

        @pl.loop(0, SEG, step=2)
        def _(j):
            pltpu.async_copy(g_hbm.at[src_at(j + 1)], rows_b, sem_b)
            pltpu.make_async_copy(g_hbm.at[src_at(j)], rows_a, sem_a).wait()
            pltpu.sync_copy(rows_a, acc_sh.at[dst_at(j)], add=True)

            @pl.when(j + 2 < SEG)
            def _():
                pltpu.async_copy(g_hbm.at[src_at(j + 2)], rows_a, sem_a)

            pltpu.make_async_copy(g_hbm.at[src_at(j + 1)], rows_b, sem_b).wait()
            pltpu.sync_copy(rows_b, acc_sh.at[dst_at(j + 1)], add=True)

    plsc.subcore_barrier()
    z0 = pl.multiple_of(s * Z, 8)

    @pl.when(c == 0)
    def _():
        pltpu.sync_copy(acc_sh.at[pl.ds(z0, Z)], out0_hbm.at[pl.ds(z0, Z)])

    @pl.when(c == 1)
    def _():
        pltpu.sync_copy(acc_sh.at[pl.ds(z0, Z)], out1_hbm.at[pl.ds(z0, Z)])


@functools.cache
def _build_sc_kernels():
    mesh = plsc.VectorSubcoreMesh(core_axis_name="c", subcore_axis_name="s",
                                  num_cores=NC, num_subcores=NS)
    deg = pl.kernel(
        _sc_degree_body,
        out_type=[jax.ShapeDtypeStruct((N_PAD,), jnp.float32),
                  jax.ShapeDtypeStruct((N_PAD,), jnp.float32)],
        mesh=mesh,
        scratch_types=[
            pltpu.VMEM((K_DEG * CHUNK,), jnp.int32),
            pltpu.VMEM((CHUNK,), jnp.float32),
            pltpu.VMEM((Z,), jnp.float32),
            pltpu.VMEM_SHARED((N_PAD,), jnp.float32),
            pltpu.SemaphoreType.DMA,
        ],
    )
    agg = pl.kernel(
        _sc_aggregate_body,
        out_type=[jax.ShapeDtypeStruct((N_PAD, D), jnp.float32),
                  jax.ShapeDtypeStruct((N_PAD, D), jnp.float32)],
        mesh=mesh,
        scratch_types=[
            pltpu.VMEM((SEG * CHUNK,), jnp.int32),
            pltpu.VMEM((SEG * CHUNK,), jnp.int32),
            pltpu.VMEM((CHUNK, D), jnp.float32),
            pltpu.VMEM((CHUNK, D), jnp.float32),
            pltpu.VMEM_SHARED((N_PAD, D), jnp.float32),
            pltpu.SemaphoreType.DMA,
            pltpu.SemaphoreType.DMA,
        ],
    )
    return deg, agg


def _sc_degree(dst_p):
    return _build_sc_kernels()[0](dst_p)


def _sc_aggregate(g, src_p, dst_p):
    return _build_sc_kernels()[1](g, src_p, dst_p)


BLK = 1000


def _tc1_body(x_ref, w_ref, p0_ref, p1_ref, g_ref, dis_ref):
    deg = p0_ref[...] + p1_ref[...] + 1.0
    dis = lax.rsqrt(deg)
    h = jnp.dot(x_ref[...], w_ref[...], preferred_element_type=jnp.float32)
    g_ref[...] = h * dis
    dis_ref[...] = dis


def _tc2_body(a0_ref, a1_ref, g_ref, dis_ref, b_ref, w_ref, g2_ref):
    dis = dis_ref[...]
    z = (a0_ref[...] + a1_ref[...] + g_ref[...]) * dis + b_ref[...]
    z = jnp.maximum(z, 0.0)
    h = jnp.dot(z, w_ref[...], preferred_element_type=jnp.float32)
    g2_ref[...] = h * dis


def _tc3_body(a0_ref, a1_ref, g_ref, dis_ref, b_ref, o_ref):
    t = (a0_ref[...] + a1_ref[...] + g_ref[...]) * dis_ref[...] + b_ref[...]
    m = jnp.max(t, axis=1, keepdims=True)
    lse = jnp.log(jnp.sum(jnp.exp(t - m), axis=1, keepdims=True))
    o_ref[...] = t - m - lse


def _row_spec(width):
    return pl.BlockSpec((BLK, width), lambda i: (i, 0))


def _full_spec(shape):
    return pl.BlockSpec(shape, lambda i: tuple(0 for _ in shape))


def _build_tc(interpret=False):
    tc1 = pl.pallas_call(
        _tc1_body,
        grid=(N // BLK,),
        in_specs=[_row_spec(D), _full_spec((D, D)), _row_spec(1), _row_spec(1)],
        out_specs=[_row_spec(D), _row_spec(1)],
        out_shape=[jax.ShapeDtypeStruct((N, D), jnp.float32),
                   jax.ShapeDtypeStruct((N, 1), jnp.float32)],
        interpret=interpret,
    )
    tc2 = pl.pallas_call(
        _tc2_body,
        grid=(N // BLK,),
        in_specs=[_row_spec(D), _row_spec(D), _row_spec(D), _row_spec(1),
                  _full_spec((1, D)), _full_spec((D, D))],
        out_specs=_row_spec(D),
        out_shape=jax.ShapeDtypeStruct((N, D), jnp.float32),
        interpret=interpret,
    )
    tc3 = pl.pallas_call(
        _tc3_body,
        grid=(N // BLK,),
        in_specs=[_row_spec(D), _row_spec(D), _row_spec(D), _row_spec(1),
                  _full_spec((1, D))],
        out_specs=_row_spec(D),
        out_shape=jax.ShapeDtypeStruct((N, D), jnp.float32),
        interpret=interpret,
    )
    return tc1, tc2, tc3


_tc1, _tc2, _tc3 = _build_tc()


def kernel(x, edge_index, W1, b1, W2, b2):
    src = edge_index[0].astype(jnp.int32)
    dst = edge_index[1].astype(jnp.int32)
    pad = E_PAD - E
    pad_idx = np.arange(pad, dtype=np.int32)
    src_pad = pad_idx % N
    dst_pad = N + pad_idx % (N_PAD - N)
    src_p = jnp.concatenate([src, jnp.asarray(src_pad)])
    dst_p = jnp.concatenate([dst, jnp.asarray(dst_pad)])

    p0, p1 = _sc_degree(dst_p)

    g1, dis = _tc1(x, W1, p0[:, None], p1[:, None])

    a0, a1 = _sc_aggregate(g1, src_p, dst_p)
    g2 = _tc2(a0, a1, g1, dis, b1[None, :], W2)

    a0, a1 = _sc_aggregate(g2, src_p, dst_p)
    return _tc3(a0, a1, g2, dis, b2[None, :])
